# Initial kernel scaffold; baseline (speedup 1.0000x reference)
import jax
import jax.numpy as jnp
from jax.experimental import pallas as pl


def kernel(features, source_intrinsics, source_extrinsics, target_K, target_E, W_depth, b_depth):
    raise NotImplementedError("write your pallas kernel here")


# speedup vs baseline: 6.7081x; 6.7081x over previous
import functools

import jax
import jax.numpy as jnp
from jax import lax
from jax.experimental import pallas as pl
from jax.experimental.pallas import tpu as pltpu

B, N, C_IN, FH, FW = 1, 6, 256, 20, 36
D_BINS, C_CTX = 64, 128
IH, IW = 320, 576
TH, TW = 20, 36
HW = FH * FW
Q = N * HW
P = TH * TW
SCALE_W = TW / IW
SCALE_H = TH / IH


def _frustum_parts():
    ds = jnp.arange(1.0, 60.0, 0.921875, dtype=jnp.float32)
    us = jnp.linspace(0.0, IW - 1.0, FW, dtype=jnp.float32)
    vs = jnp.linspace(0.0, IH - 1.0, FH, dtype=jnp.float32)
    u_hw = jnp.tile(us, (FH,))
    v_hw = jnp.repeat(vs, FW)
    return u_hw, v_hw, ds


def _precise_div(a, b):
    r = 1.0 / b
    r = r * (2.0 - b * r)
    q = a * r
    return q + r * (a - b * q)


def _lift_geom_body(geom_ref, x_ref, w_ref, b_ref, uv_ref, ds_ref,
                    idx_ref, dpm_ref, ctx_ref):
    n = pl.program_id(0)
    x = x_ref[0]
    w = w_ref[...]
    y = lax.dot_general(x, w, (((0,), (1,)), ((), ())),
                        preferred_element_type=jnp.float32,
                        precision=lax.Precision.HIGHEST)
    y = y + b_ref[...]
    logits = y[:, :D_BINS]
    m = jnp.max(logits, axis=1, keepdims=True)
    e = jnp.exp(logits - m)
    dp = _precise_div(e, jnp.sum(e, axis=1, keepdims=True))
    ctx = y[:, D_BINS:]

    u = uv_ref[:, 0:1]
    v = uv_ref[:, 1:2]
    d = ds_ref[...]
    bf = lambda a: a.astype(jnp.bfloat16).astype(jnp.float32)
    ph0 = bf(u * d)
    ph1 = bf(v * d)
    ph2 = bf(jnp.broadcast_to(d, ph0.shape))

    g = lambda k: geom_ref[n, k]
    c0 = g(0) * ph0 + g(1) * ph1 + g(2) * ph2
    c1 = g(3) * ph0 + g(4) * ph1 + g(5) * ph2
    c2 = g(6) * ph0 + g(7) * ph1 + g(8) * ph2
    b0, b1, b2 = bf(c0), bf(c1), bf(c2)
    e0 = g(9) * b0 + g(10) * b1 + g(11) * b2 + g(18)
    e1 = g(12) * b0 + g(13) * b1 + g(14) * b2 + g(19)
    e2 = g(15) * b0 + g(16) * b1 + g(17) * b2 + g(20)
    b0, b1, b2 = bf(e0), bf(e1), bf(e2)
    t0 = g(21) * b0 + g(22) * b1 + g(23) * b2 + g(30)
    t1 = g(24) * b0 + g(25) * b1 + g(26) * b2 + g(31)
    t2 = g(27) * b0 + g(28) * b1 + g(29) * b2 + g(32)
    z = jnp.maximum(t2, 0.1)
    uu = _precise_div(g(33) * t0, z) + g(35)
    vv = _precise_div(g(34) * t1, z) + g(36)
    uf = (uu * SCALE_W).astype(jnp.int32)
    vf = (vv * SCALE_H).astype(jnp.int32)
    valid = (t2 > 0.1) & (uf >= 0) & (uf < TW) & (vf >= 0) & (vf < TH)
    idx = jnp.where(valid, vf * TW + uf, 0)
    dpm = jnp.where(valid, dp, 0.0)

    idx_ref[...] = idx
    dpm_ref[...] = dpm
    ctx_ref[0] = ctx


def _lift_geom(geom, x, w, b2, uv, ds2):
    return pl.pallas_call(
        _lift_geom_body,
        grid=(N,),
        in_specs=[
            pl.BlockSpec(memory_space=pltpu.SMEM),
            pl.BlockSpec((1, C_IN, HW), lambda n: (n, 0, 0)),
            pl.BlockSpec((D_BINS + C_CTX, C_IN), lambda n: (0, 0)),
            pl.BlockSpec((1, D_BINS + C_CTX), lambda n: (0, 0)),
            pl.BlockSpec((HW, 2), lambda n: (0, 0)),
            pl.BlockSpec((1, D_BINS), lambda n: (0, 0)),
        ],
        out_specs=[
            pl.BlockSpec((HW, D_BINS), lambda n: (n, 0)),
            pl.BlockSpec((HW, D_BINS), lambda n: (n, 0)),
            pl.BlockSpec((1, HW, C_CTX), lambda n: (n, 0, 0)),
        ],
        out_shape=[
            jax.ShapeDtypeStruct((Q, D_BINS), jnp.int32),
            jax.ShapeDtypeStruct((Q, D_BINS), jnp.float32),
            jax.ShapeDtypeStruct((N, HW, C_CTX), jnp.float32),
        ],
    )(geom, x, w, b2, uv, ds2)


def _onehot_a_body(idx_ref, dpm_ref, a_ref):
    p_row = lax.broadcasted_iota(jnp.int32, (1, P), 1)
    acc = jnp.zeros((HW, P), jnp.float32)
    for d in range(D_BINS):
        hit = (idx_ref[:, d:d + 1] == p_row)
        acc = acc + jnp.where(hit, dpm_ref[:, d:d + 1], 0.0)
    a_ref[...] = acc


def _onehot_a(idx, dpm):
    return pl.pallas_call(
        _onehot_a_body,
        grid=(N,),
        in_specs=[
            pl.BlockSpec((HW, D_BINS), lambda n: (n, 0)),
            pl.BlockSpec((HW, D_BINS), lambda n: (n, 0)),
        ],
        out_specs=pl.BlockSpec((HW, P), lambda n: (n, 0)),
        out_shape=jax.ShapeDtypeStruct((Q, P), jnp.float32),
    )(idx, dpm)


def _contract_body(ctx_ref, a_ref, out_ref):
    n = pl.program_id(0)
    part = lax.dot_general(ctx_ref[0], a_ref[...], (((0,), (0,)), ((), ())),
                           preferred_element_type=jnp.float32,
                           precision=lax.Precision.HIGHEST)
    @pl.when(n == 0)
    def _():
        out_ref[...] = part

    @pl.when(n > 0)
    def _():
        out_ref[...] += part


def _contract(ctx, a):
    return pl.pallas_call(
        _contract_body,
        grid=(N,),
        in_specs=[
            pl.BlockSpec((1, HW, C_CTX), lambda n: (n, 0, 0)),
            pl.BlockSpec((HW, P), lambda n: (n, 0)),
        ],
        out_specs=pl.BlockSpec((C_CTX, P), lambda n: (0, 0)),
        out_shape=jax.ShapeDtypeStruct((C_CTX, P), jnp.float32),
    )(ctx, a)


def kernel(features, source_intrinsics, source_extrinsics, target_K, target_E,
           W_depth, b_depth):
    x = features.reshape(N, C_IN, HW)
    K_inv = jnp.linalg.inv(source_intrinsics)
    R = source_extrinsics[:, :, :3, :3]
    t = source_extrinsics[:, :, :3, 3]
    R_tgt = target_E[:, :3, :3]
    t_tgt = target_E[:, :3, 3]
    fx = target_K[:, 0, 0]
    fy = target_K[:, 1, 1]
    cx = target_K[:, 0, 2]
    cy = target_K[:, 1, 2]

    def bfr(a):
        u = lax.bitcast_convert_type(a, jnp.uint32)
        u = u + jnp.uint32(0x7FFF) + ((u >> 16) & jnp.uint32(1))
        u = u & jnp.uint32(0xFFFF0000)
        return lax.bitcast_convert_type(u, jnp.float32)

    geom = jnp.concatenate([
        bfr(K_inv[0]).reshape(N, 9),
        bfr(R[0]).reshape(N, 9),
        t[0].reshape(N, 3),
        jnp.broadcast_to(bfr(R_tgt[0]).reshape(1, 9), (N, 9)),
        jnp.broadcast_to(t_tgt[0].reshape(1, 3), (N, 3)),
        jnp.broadcast_to(jnp.stack([fx[0], fy[0], cx[0], cy[0]]).reshape(1, 4),
                         (N, 4)),
    ], axis=1)

    u_hw, v_hw, ds = _frustum_parts()
    uv = jnp.stack([u_hw, v_hw], axis=1)
    ds2 = ds.reshape(1, D_BINS)
    b2 = b_depth.reshape(1, D_BINS + C_CTX)

    idx, dpm, ctx = _lift_geom(geom, x, W_depth, b2, uv, ds2)
    a = _onehot_a(idx, dpm)
    out = _contract(ctx, a)
    return out.reshape(1, C_CTX, TH, TW)
